# baseline (speedup 1.0000x reference)
import functools

import jax
import jax.numpy as jnp
from jax import lax
from jax.experimental import pallas as pl
from jax.experimental.pallas import tpu as pltpu
from jax.experimental.pallas import tpu_sc as plsc

D = 32
B = 4096
L = 200
CH = 40
LP = 208
NC = 2
NS = 16
NW = NC * NS
BPW = B // NW
NBUF = 2
GROUPS = BPW // NBUF


_sc_mesh = plsc.VectorSubcoreMesh(core_axis_name="c", subcore_axis_name="s")


@functools.partial(
    pl.kernel,
    out_type=jax.ShapeDtypeStruct((B, D), jnp.float32),
    mesh=_sc_mesh,
    scratch_types=[
        pltpu.VMEM((BPW * L,), jnp.int32),
        pltpu.VMEM((BPW * LP,), jnp.int32),
        pltpu.VMEM((NBUF, L, 128), jnp.float32),
        pltpu.VMEM((BPW, D), jnp.float32),
    ] + [pltpu.SemaphoreType.DMA] * NBUF,
    compiler_params=pltpu.CompilerParams(use_tc_tiling_on_sc=True),
)
def _gather_pool(xq_hbm, qoff_hbm, t128_hbm, pooled_hbm,
                 xq_v, qoff_v, buf_v, pooled_v, *sems):
    wid = lax.axis_index("s") * NC + lax.axis_index("c")

    pltpu.sync_copy(xq_hbm.at[pl.ds(wid * (BPW * L), BPW * L)], xq_v)
    pltpu.sync_copy(qoff_hbm.at[pl.ds(wid * (BPW * LP), BPW * LP)], qoff_v)

    def start_row(r, slot):
        for c in range(5):
            pltpu.make_async_copy(
                t128_hbm.at[xq_v.at[pl.ds(r * L + c * CH, CH)]],
                buf_v.at[slot, pl.ds(c * CH, CH)], sems[slot]).start()

    def wait_row(slot):
        for c in range(5):
            pltpu.make_async_copy(
                t128_hbm.at[xq_v.at[pl.ds(c * CH, CH)]],
                buf_v.at[slot, pl.ds(c * CH, CH)], sems[slot]).wait()

    for i in range(NBUF):
        start_row(i, i)

    zero = jnp.zeros((16,), jnp.float32)

    @pl.loop(0, GROUPS)
    def _group(g):
        for i in range(NBUF):
            r = g * NBUF + i
            wait_row(i)

            @pl.loop(0, 12, init_carry=(zero, zero, zero, zero))
            def blocks(t, carry):
                a0, a1, c0, c1 = carry
                qv = qoff_v[pl.ds(r * LP + t * 16, 16)]
                for l in range(0, 16, 2):
                    j = t * 16 + l
                    o0 = qv[l]
                    o1 = qv[l + 1]
                    a0 = a0 + buf_v[i, j, pl.ds(o0, 16)]
                    a1 = a1 + buf_v[i, j, pl.ds(o0 + 16, 16)]
                    c0 = c0 + buf_v[i, j + 1, pl.ds(o1, 16)]
                    c1 = c1 + buf_v[i, j + 1, pl.ds(o1 + 16, 16)]
                return a0, a1, c0, c1

            a0, a1, c0, c1 = blocks
            qv = qoff_v[pl.ds(r * LP + 192, 16)]
            for l in range(0, 8, 2):
                o0 = qv[l]
                o1 = qv[l + 1]
                a0 = a0 + buf_v[i, 192 + l, pl.ds(o0, 16)]
                a1 = a1 + buf_v[i, 192 + l, pl.ds(o0 + 16, 16)]
                c0 = c0 + buf_v[i, 193 + l, pl.ds(o1, 16)]
                c1 = c1 + buf_v[i, 193 + l, pl.ds(o1 + 16, 16)]

            @pl.when(g < GROUPS - 1)
            def _refill():
                start_row(r + NBUF, i)

            pooled_v[r, pl.ds(0, 16)] = a0 + c0
            pooled_v[r, pl.ds(16, 16)] = a1 + c1

    pltpu.sync_copy(pooled_v, pooled_hbm.at[pl.ds(wid * BPW, BPW), :])


def _mlp_body(p_ref, w1_ref, b1_ref, w2t_ref, b2_ref, o_ref):
    p = p_ref[...] * (1.0 / L)
    h = jnp.maximum(
        jnp.dot(p, w1_ref[...], preferred_element_type=jnp.float32)
        + b1_ref[...], 0.0)
    o = jnp.sum(h * w2t_ref[...], axis=1, keepdims=True) + b2_ref[...]
    o_ref[...] = 1.0 / (1.0 + jnp.exp(-o))


def kernel(x, table, W1, b1, W2, b2):
    xi = x.astype(jnp.int32)
    xq = (xi >> 2).reshape(-1)
    qoff = jnp.pad((xi & 3) << 5, ((0, 0), (0, LP - L))).reshape(-1)
    t128 = table.reshape(250000, 128)
    pooled = _gather_pool(xq, qoff, t128)
    out = pl.pallas_call(
        _mlp_body,
        out_shape=jax.ShapeDtypeStruct((B, 1), jnp.float32),
    )(pooled, W1, b1.reshape(1, 16), W2.reshape(1, 16), b2.reshape(1, 1))
    return out
